# baseline (speedup 1.0000x reference)
import functools

import jax
import jax.numpy as jnp
from jax.experimental import pallas as pl
from jax.experimental.pallas import tpu as pltpu


def _video_encode_kernel(x_ref, w_ref, mask_ref, out_ref):
    x = x_ref[0].astype(jnp.bfloat16)
    ft = jnp.dot(w_ref[...], x, preferred_element_type=jnp.float32)
    ssum = jnp.sum(ft * ft, axis=0, keepdims=True)
    m = mask_ref[0]
    out_ref[...] = (ft * (jax.lax.rsqrt(ssum) * m))[None]


def _head_kernel(vfn_ref, tok_ref, emb_ref, pos_ref, wt_ref, ls_ref,
                 loss_ref, *, L, inv_b):
    pooled = jnp.sum(vfn_ref[...], axis=0)
    pinv = jax.lax.rsqrt(jnp.sum(pooled * pooled, axis=0, keepdims=True))
    vf = pooled * pinv

    tok = tok_ref[...]
    b, v = tok.shape[0], emb_ref.shape[0]
    viota = jax.lax.broadcasted_iota(jnp.int32, (b, v), 1)
    counts = jnp.zeros((b, v), jnp.float32)
    for l in range(L):
        counts += (viota == tok[:, l][:, None]).astype(jnp.float32)
    xt = jnp.dot((counts * (1.0 / L)).astype(jnp.bfloat16),
                 emb_ref[...].astype(jnp.bfloat16),
                 preferred_element_type=jnp.float32)
    xt += jnp.mean(pos_ref[0:L], axis=0, keepdims=True)
    seq = jnp.dot(xt.astype(jnp.bfloat16),
                  wt_ref[...].astype(jnp.bfloat16),
                  preferred_element_type=jnp.float32)
    tinv = jax.lax.rsqrt(jnp.sum(seq * seq, axis=-1, keepdims=True))
    tn = seq * tinv

    scale = jnp.exp(ls_ref[0, 0])
    sim = scale * jnp.dot(tn, vf, preferred_element_type=jnp.float32)
    r = jax.lax.broadcasted_iota(jnp.int32, (b, b), 0)
    c = jax.lax.broadcasted_iota(jnp.int32, (b, b), 1)
    diag = jnp.sum(jnp.where(r == c, sim, 0.0))
    mr = jnp.max(sim, axis=1, keepdims=True)
    racc = jnp.sum(jnp.log(jnp.sum(jnp.exp(sim - mr), axis=1, keepdims=True)) + mr)
    mc = jnp.max(sim, axis=0, keepdims=True)
    cacc = jnp.sum(jnp.log(jnp.sum(jnp.exp(sim - mc), axis=0, keepdims=True)) + mc)
    loss = ((racc - diag) + (cacc - diag)) * (0.5 * inv_b)
    loss_ref[...] = jnp.reshape(loss, (1, 1))


def kernel(tok_emb, pos_emb, w_text, w_patch, logit_scale,
           text_input, video, video_mask):
    B, L = text_input.shape
    _, T, C, H, W = video.shape
    D = w_patch.shape[1]
    Kt = tok_emb.shape[1]
    P = int(round((w_patch.shape[0] // C) ** 0.5))
    nh, nw = H // P, W // P
    CHW = C * H * W

    wt4 = ((w_patch.T).reshape(D, C, P, P) * (1.0 / (nh * nw))
           ).astype(jnp.bfloat16)
    wmap_t = jnp.broadcast_to(
        wt4[:, :, None, :, None, :], (D, C, nh, P, nw, P)).reshape(D, CHW)

    xs = video.transpose(1, 2, 3, 4, 0).reshape(T, CHW, B)
    mask_t = video_mask.astype(jnp.float32).T.reshape(T, 1, B)

    vfn = pl.pallas_call(
        _video_encode_kernel,
        out_shape=jax.ShapeDtypeStruct((T, D, B), jnp.float32),
        grid_spec=pltpu.PrefetchScalarGridSpec(
            num_scalar_prefetch=0,
            grid=(T,),
            in_specs=[pl.BlockSpec((1, CHW, B), lambda t: (t, 0, 0)),
                      pl.BlockSpec((D, CHW), lambda t: (0, 0)),
                      pl.BlockSpec((1, 1, B), lambda t: (t, 0, 0))],
            out_specs=pl.BlockSpec((1, D, B), lambda t: (t, 0, 0))),
        compiler_params=pltpu.CompilerParams(
            dimension_semantics=("arbitrary",),
            vmem_limit_bytes=64 * 1024 * 1024),
        cost_estimate=pl.CostEstimate(
            flops=2 * T * CHW * B * D,
            transcendentals=0,
            bytes_accessed=T * CHW * B * 4 + D * CHW * 2 + T * B * D * 4),
    )(xs, wmap_t, mask_t)

    V = tok_emb.shape[0]
    loss = pl.pallas_call(
        functools.partial(_head_kernel, L=L, inv_b=1.0 / B),
        out_shape=jax.ShapeDtypeStruct((1, 1), jnp.float32),
        grid_spec=pltpu.PrefetchScalarGridSpec(
            num_scalar_prefetch=0,
            grid=(1,),
            in_specs=[pl.BlockSpec((T, D, B), lambda i: (0, 0, 0)),
                      pl.BlockSpec((B, L), lambda i: (0, 0)),
                      pl.BlockSpec((V, Kt), lambda i: (0, 0)),
                      pl.BlockSpec(pos_emb.shape, lambda i: (0, 0)),
                      pl.BlockSpec((Kt, D), lambda i: (0, 0)),
                      pl.BlockSpec((1, 1), lambda i: (0, 0))],
            out_specs=pl.BlockSpec((1, 1), lambda i: (0, 0))),
        compiler_params=pltpu.CompilerParams(
            dimension_semantics=("arbitrary",)),
    )(vfn, text_input, tok_emb, pos_emb, w_text,
      logit_scale.reshape(1, 1))
    return loss[0, 0]


# speedup vs baseline: 1.0984x; 1.0984x over previous
import functools

import jax
import jax.numpy as jnp
from jax.experimental import pallas as pl
from jax.experimental.pallas import tpu as pltpu


def _video_encode_kernel(x_ref, w_ref, mask_ref, out_ref, *, C, P, nh, nw):
    x = x_ref[0]
    bl = x.shape[-1]
    s1 = jnp.sum(x.reshape(C * nh * P, nw, P, bl), axis=1)
    s2 = jnp.sum(s1.reshape(C, nh, P, P, bl), axis=1)
    pp = s2.reshape(C * P * P, bl).astype(jnp.bfloat16)
    ft = jnp.dot(w_ref[...], pp, preferred_element_type=jnp.float32)
    ssum = jnp.sum(ft * ft, axis=0, keepdims=True)
    m = mask_ref[0]
    out_ref[...] = (ft * (jax.lax.rsqrt(ssum) * m))[None]


def _head_kernel(vfn_ref, tok_ref, emb_ref, pos_ref, wt_ref, ls_ref,
                 loss_ref, *, L, inv_b):
    pooled = jnp.sum(vfn_ref[...], axis=0)
    pinv = jax.lax.rsqrt(jnp.sum(pooled * pooled, axis=0, keepdims=True))
    vf = pooled * pinv

    tok = tok_ref[...]
    b, v = tok.shape[0], emb_ref.shape[0]
    viota = jax.lax.broadcasted_iota(jnp.int32, (b, v), 1)
    counts = jnp.zeros((b, v), jnp.float32)
    for l in range(L):
        counts += (viota == tok[:, l][:, None]).astype(jnp.float32)
    xt = jnp.dot((counts * (1.0 / L)).astype(jnp.bfloat16),
                 emb_ref[...].astype(jnp.bfloat16),
                 preferred_element_type=jnp.float32)
    xt += jnp.mean(pos_ref[0:L], axis=0, keepdims=True)
    seq = jnp.dot(xt.astype(jnp.bfloat16),
                  wt_ref[...].astype(jnp.bfloat16),
                  preferred_element_type=jnp.float32)
    tinv = jax.lax.rsqrt(jnp.sum(seq * seq, axis=-1, keepdims=True))
    tn = seq * tinv

    scale = jnp.exp(ls_ref[0, 0])
    sim = scale * jnp.dot(tn, vf, preferred_element_type=jnp.float32)
    r = jax.lax.broadcasted_iota(jnp.int32, (b, b), 0)
    c = jax.lax.broadcasted_iota(jnp.int32, (b, b), 1)
    diag = jnp.sum(jnp.where(r == c, sim, 0.0))
    mr = jnp.max(sim, axis=1, keepdims=True)
    racc = jnp.sum(jnp.log(jnp.sum(jnp.exp(sim - mr), axis=1, keepdims=True)) + mr)
    mc = jnp.max(sim, axis=0, keepdims=True)
    cacc = jnp.sum(jnp.log(jnp.sum(jnp.exp(sim - mc), axis=0, keepdims=True)) + mc)
    loss = ((racc - diag) + (cacc - diag)) * (0.5 * inv_b)
    loss_ref[...] = jnp.reshape(loss, (1, 1))


def kernel(tok_emb, pos_emb, w_text, w_patch, logit_scale,
           text_input, video, video_mask):
    B, L = text_input.shape
    _, T, C, H, W = video.shape
    D = w_patch.shape[1]
    Kt = tok_emb.shape[1]
    P = int(round((w_patch.shape[0] // C) ** 0.5))
    nh, nw = H // P, W // P
    CHW = C * H * W

    wp_t = ((w_patch.T) * (1.0 / (nh * nw))).astype(jnp.bfloat16)

    xs = video.transpose(1, 2, 3, 4, 0).reshape(T, CHW, B)
    mask_t = video_mask.astype(jnp.float32).T.reshape(T, 1, B)

    vfn = pl.pallas_call(
        functools.partial(_video_encode_kernel, C=C, P=P, nh=nh, nw=nw),
        out_shape=jax.ShapeDtypeStruct((T, D, B), jnp.float32),
        grid_spec=pltpu.PrefetchScalarGridSpec(
            num_scalar_prefetch=0,
            grid=(T,),
            in_specs=[pl.BlockSpec((1, CHW, B), lambda t: (t, 0, 0)),
                      pl.BlockSpec((D, C * P * P), lambda t: (0, 0)),
                      pl.BlockSpec((1, 1, B), lambda t: (t, 0, 0))],
            out_specs=pl.BlockSpec((1, D, B), lambda t: (t, 0, 0))),
        compiler_params=pltpu.CompilerParams(
            dimension_semantics=("arbitrary",),
            vmem_limit_bytes=64 * 1024 * 1024),
        cost_estimate=pl.CostEstimate(
            flops=T * CHW * B + 2 * T * C * P * P * B * D,
            transcendentals=0,
            bytes_accessed=T * CHW * B * 4 + D * C * P * P * 2 + T * B * D * 4),
    )(xs, wp_t, mask_t)

    V = tok_emb.shape[0]
    loss = pl.pallas_call(
        functools.partial(_head_kernel, L=L, inv_b=1.0 / B),
        out_shape=jax.ShapeDtypeStruct((1, 1), jnp.float32),
        grid_spec=pltpu.PrefetchScalarGridSpec(
            num_scalar_prefetch=0,
            grid=(1,),
            in_specs=[pl.BlockSpec((T, D, B), lambda i: (0, 0, 0)),
                      pl.BlockSpec((B, L), lambda i: (0, 0)),
                      pl.BlockSpec((V, Kt), lambda i: (0, 0)),
                      pl.BlockSpec(pos_emb.shape, lambda i: (0, 0)),
                      pl.BlockSpec((Kt, D), lambda i: (0, 0)),
                      pl.BlockSpec((1, 1), lambda i: (0, 0))],
            out_specs=pl.BlockSpec((1, 1), lambda i: (0, 0))),
        compiler_params=pltpu.CompilerParams(
            dimension_semantics=("arbitrary",)),
    )(vfn, text_input, tok_emb, pos_emb, w_text,
      logit_scale.reshape(1, 1))
    return loss[0, 0]
